# Initial kernel scaffold; baseline (speedup 1.0000x reference)
import functools

import jax
import jax.numpy as jnp
from jax import lax
from jax.experimental import pallas as pl
from jax.experimental.pallas import tpu as pltpu
from jax.experimental.pallas import tpu_sc as plsc

N = 10000
NP = 10240
E = 160000
D = 256
HALF = 128
C_OUT = 8
BLK = 256
NBLK = NP // BLK
NSUB = 16
CH = 125
CPW = E // (NSUB * CH)
RPS = NP // NSUB

_MESH = plsc.VectorSubcoreMesh(core_axis_name="c", subcore_axis_name="s")


@functools.partial(
    pl.kernel,
    mesh=_MESH,
    out_type=[
        jax.ShapeDtypeStruct((NP, 16), jnp.float32),
        jax.ShapeDtypeStruct((NP, 16), jnp.float32),
    ],
    scratch_types=[
        pltpu.VMEM((CPW, CH), jnp.int32),
        pltpu.VMEM((CH, 16), jnp.float32),
        pltpu.VMEM_SHARED((NP, 16), jnp.float32),
    ],
)
def _sc_degrees(e_ref, ones_ref, z_ref, dego_ref, degi_ref, idxs_v, ones_v, deg_sh):
    c = lax.axis_index("c")
    s = lax.axis_index("s")
    pltpu.sync_copy(z_ref, deg_sh.at[pl.ds(s * RPS, RPS)])
    pltpu.sync_copy(ones_ref, ones_v)
    pltpu.sync_copy(e_ref.at[c * NSUB + s], idxs_v)
    plsc.subcore_barrier()

    def chunk(g, carry):
        pltpu.sync_copy(ones_v, deg_sh.at[idxs_v.at[g]], add=True)
        return carry

    lax.fori_loop(0, CPW, chunk, 0)
    plsc.subcore_barrier()
    sl = pl.ds(s * RPS, RPS)
    pl.when(c == 0)(lambda: pltpu.sync_copy(deg_sh.at[sl], dego_ref.at[sl]))
    pl.when(c == 1)(lambda: pltpu.sync_copy(deg_sh.at[sl], degi_ref.at[sl]))


@functools.partial(
    pl.kernel,
    mesh=_MESH,
    out_type=[
        jax.ShapeDtypeStruct((NP, HALF), jnp.float32),
        jax.ShapeDtypeStruct((NP, HALF), jnp.float32),
    ],
    scratch_types=[
        pltpu.VMEM((CPW, CH), jnp.int32),
        pltpu.VMEM((CPW, CH), jnp.int32),
        pltpu.VMEM((CH, HALF), jnp.float32),
        pltpu.VMEM_SHARED((NP, HALF), jnp.float32),
        pltpu.SemaphoreType.DMA,
    ],
)
def _sc_segsum(e_ref, p0_ref, p1_ref, z_ref, out0_ref, out1_ref,
               sidx_v, didx_v, rows_v, agg_sh, sem):
    c = lax.axis_index("c")
    s = lax.axis_index("s")
    pltpu.sync_copy(z_ref, agg_sh.at[pl.ds(s * RPS, RPS)])
    pltpu.sync_copy(e_ref.at[s], sidx_v)
    pltpu.sync_copy(e_ref.at[NSUB + s], didx_v)
    plsc.subcore_barrier()

    def run(p_ref):
        def chunk(g, carry):
            pltpu.async_copy(p_ref.at[sidx_v.at[g]], rows_v, sem).wait()
            pltpu.sync_copy(rows_v, agg_sh.at[didx_v.at[g]], add=True)
            return carry

        lax.fori_loop(0, CPW, chunk, 0)

    pl.when(c == 0)(lambda: run(p0_ref))
    pl.when(c == 1)(lambda: run(p1_ref))
    plsc.subcore_barrier()
    sl = pl.ds(s * RPS, RPS)
    pl.when(c == 0)(lambda: pltpu.sync_copy(agg_sh.at[sl], out0_ref.at[sl]))
    pl.when(c == 1)(lambda: pltpu.sync_copy(agg_sh.at[sl], out1_ref.at[sl]))


def _norm_col(deg_ref):
    d = deg_ref[:, 0:1]
    return jnp.where(d > 0.0, 1.0 / jnp.sqrt(jnp.maximum(d, 1.0)), 0.0)


def _dot(a, b):
    return lax.dot_general(a, b, (((1,), (0,)), ((), ())),
                           precision=lax.Precision.HIGHEST,
                           preferred_element_type=jnp.float32)


def _mm1_body(x_ref, dego_ref, w_ref, out0_ref, out1_ref):
    ns = _norm_col(dego_ref)
    p = _dot(x_ref[...] * ns, w_ref[...])
    out0_ref[...] = p[:, :HALF]
    out1_ref[...] = p[:, HALF:]


def _mm2_body(a0_ref, a1_ref, degi_ref, dego_ref, b_ref, w_ref, out0_ref, out1_ref):
    nd = _norm_col(degi_ref)
    ns = _norm_col(dego_ref)
    h0 = jnp.tanh(a0_ref[...] * nd + b_ref[0:1, :]) * ns
    h1 = jnp.tanh(a1_ref[...] * nd + b_ref[1:2, :]) * ns
    w = w_ref[...]
    p = _dot(h0, w[:HALF, :]) + _dot(h1, w[HALF:, :])
    out0_ref[...] = p[:, :HALF]
    out1_ref[...] = p[:, HALF:]


def _final_body(a0_ref, a1_ref, degi_ref, b_ref, wc_ref, bc_ref, out_ref, acc_ref):
    i = pl.program_id(0)
    nd = _norm_col(degi_ref)
    h0 = jnp.tanh(a0_ref[...] * nd + b_ref[0:1, :])
    h1 = jnp.tanh(a1_ref[...] * nd + b_ref[1:2, :])
    row = lax.broadcasted_iota(jnp.int32, (BLK, 1), 0) + i * BLK
    m = (row < N).astype(jnp.float32)
    s0 = jnp.sum(h0 * m, axis=0, keepdims=True)
    s1 = jnp.sum(h1 * m, axis=0, keepdims=True)

    @pl.when(i == 0)
    def _():
        acc_ref[...] = jnp.zeros_like(acc_ref)

    acc_ref[0:1, :] += s0
    acc_ref[1:2, :] += s1

    @pl.when(i == NBLK - 1)
    def _():
        hg0 = jnp.tanh(acc_ref[0:1, :] * (1.0 / N))
        hg1 = jnp.tanh(acc_ref[1:2, :] * (1.0 / N))
        out_ref[...] = _dot(hg0, wc_ref[0, :, :]) + _dot(hg1, wc_ref[1, :, :]) + bc_ref[...]


def _mm1(x_pad, dego16, W0):
    return pl.pallas_call(
        _mm1_body,
        grid=(NBLK,),
        in_specs=[
            pl.BlockSpec((BLK, D), lambda i: (i, 0)),
            pl.BlockSpec((BLK, 16), lambda i: (i, 0)),
            pl.BlockSpec((D, D), lambda i: (0, 0)),
        ],
        out_specs=[pl.BlockSpec((BLK, HALF), lambda i: (i, 0))] * 2,
        out_shape=[jax.ShapeDtypeStruct((NP, HALF), jnp.float32)] * 2,
    )(x_pad, dego16, W0)


def _mm2(a0, a1, degi16, dego16, br, W):
    return pl.pallas_call(
        _mm2_body,
        grid=(NBLK,),
        in_specs=[
            pl.BlockSpec((BLK, HALF), lambda i: (i, 0)),
            pl.BlockSpec((BLK, HALF), lambda i: (i, 0)),
            pl.BlockSpec((BLK, 16), lambda i: (i, 0)),
            pl.BlockSpec((BLK, 16), lambda i: (i, 0)),
            pl.BlockSpec((2, HALF), lambda i: (0, 0)),
            pl.BlockSpec((D, D), lambda i: (0, 0)),
        ],
        out_specs=[pl.BlockSpec((BLK, HALF), lambda i: (i, 0))] * 2,
        out_shape=[jax.ShapeDtypeStruct((NP, HALF), jnp.float32)] * 2,
    )(a0, a1, degi16, dego16, br, W)


def _final(a0, a1, degi16, br, wc_r, bc_r):
    return pl.pallas_call(
        _final_body,
        grid=(NBLK,),
        in_specs=[
            pl.BlockSpec((BLK, HALF), lambda i: (i, 0)),
            pl.BlockSpec((BLK, HALF), lambda i: (i, 0)),
            pl.BlockSpec((BLK, 16), lambda i: (i, 0)),
            pl.BlockSpec((2, HALF), lambda i: (0, 0)),
            pl.BlockSpec((2, HALF, HALF), lambda i: (0, 0, 0)),
            pl.BlockSpec((1, HALF), lambda i: (0, 0)),
        ],
        out_specs=pl.BlockSpec((1, HALF), lambda i: (0, 0)),
        out_shape=jax.ShapeDtypeStruct((1, HALF), jnp.float32),
        scratch_shapes=[pltpu.VMEM((2, HALF), jnp.float32)],
    )(a0, a1, degi16, br, wc_r, bc_r)


def kernel(x, edge_index, W0, b0, W1, b1, W2, b2, Wc, bc):
    x_pad = jnp.pad(x, ((0, NP - N), (0, 0)))
    e32 = edge_index.reshape(2 * NSUB, CPW, CH)

    ones16 = jnp.ones((CH, 16), jnp.float32)
    z16 = jnp.zeros((RPS, 16), jnp.float32)
    z128 = jnp.zeros((RPS, HALF), jnp.float32)

    dego16, degi16 = _sc_degrees(e32, ones16, z16)

    b0r = b0.reshape(2, HALF)
    b1r = b1.reshape(2, HALF)
    b2r = b2.reshape(2, HALF)
    wc_r = jnp.pad(Wc, ((0, 0), (0, HALF - C_OUT))).reshape(2, HALF, HALF)
    bc_r = jnp.pad(bc, (0, HALF - C_OUT)).reshape(1, HALF)

    p0, p1 = _mm1(x_pad, dego16, W0)
    a0, a1 = _sc_segsum(e32, p0, p1, z128)
    p0, p1 = _mm2(a0, a1, degi16, dego16, b0r, W1)
    a0, a1 = _sc_segsum(e32, p0, p1, z128)
    p0, p1 = _mm2(a0, a1, degi16, dego16, b1r, W2)
    a0, a1 = _sc_segsum(e32, p0, p1, z128)

    out = _final(a0, a1, degi16, b2r, wc_r, bc_r)
    return out[:, :C_OUT]


# speedup vs baseline: 4.5239x; 4.5239x over previous
import functools

import jax
import jax.numpy as jnp
from jax import lax
from jax.experimental import pallas as pl
from jax.experimental.pallas import tpu as pltpu
from jax.experimental.pallas import tpu_sc as plsc

N = 10000
NP = 10240
E = 160000
D = 256
HALF = 128
C_OUT = 8
BLK = 256
NBLK = NP // BLK
NSUB = 16
CH = 125
CPW = E // (NSUB * CH)
RPS = NP // NSUB


def _sc_degrees_body(e_ref, ones_ref, z_ref, deg_ref, idxs_v, ones_v, deg_sh):
    c = lax.axis_index("c")
    s = lax.axis_index("s")
    pltpu.sync_copy(z_ref, deg_sh.at[pl.ds(s * RPS, RPS)])
    pltpu.sync_copy(ones_ref, ones_v)
    pltpu.sync_copy(e_ref.at[c * NSUB + s], idxs_v)
    plsc.subcore_barrier()

    def chunk(g, carry):
        pltpu.sync_copy(ones_v, deg_sh.at[idxs_v.at[g]], add=True)
        return carry

    lax.fori_loop(0, CPW, chunk, 0)
    plsc.subcore_barrier()
    sl = pl.ds(s * RPS, RPS)
    pltpu.sync_copy(deg_sh.at[sl], deg_ref.at[c].at[sl])


def _sc_segsum_body(e_ref, p0_ref, p1_ref, z_ref, out0_ref, out1_ref,
                    sidx_v, didx_v, rows_v, agg_sh, sem):
    c = lax.axis_index("c")
    s = lax.axis_index("s")
    pltpu.sync_copy(z_ref, agg_sh.at[pl.ds(s * RPS, RPS)])
    pltpu.sync_copy(e_ref.at[s], sidx_v)
    pltpu.sync_copy(e_ref.at[NSUB + s], didx_v)
    plsc.subcore_barrier()

    def run(p_ref):
        def chunk(g, carry):
            pltpu.async_copy(p_ref.at[sidx_v.at[g]], rows_v, sem).wait()
            pltpu.sync_copy(rows_v, agg_sh.at[didx_v.at[g]], add=True)
            return carry

        lax.fori_loop(0, CPW, chunk, 0)

    pl.when(c == 0)(lambda: run(p0_ref))
    pl.when(c == 1)(lambda: run(p1_ref))
    plsc.subcore_barrier()
    sl = pl.ds(s * RPS, RPS)
    pl.when(c == 0)(lambda: pltpu.sync_copy(agg_sh.at[sl], out0_ref.at[sl]))
    pl.when(c == 1)(lambda: pltpu.sync_copy(agg_sh.at[sl], out1_ref.at[sl]))


@functools.cache
def _sc_kernels():
    mesh = plsc.VectorSubcoreMesh(core_axis_name="c", subcore_axis_name="s")
    degrees = pl.kernel(
        _sc_degrees_body,
        mesh=mesh,
        out_type=jax.ShapeDtypeStruct((2, NP, HALF), jnp.float32),
        scratch_types=[
            pltpu.VMEM((CPW, CH), jnp.int32),
            pltpu.VMEM((CH, HALF), jnp.float32),
            pltpu.VMEM_SHARED((NP, HALF), jnp.float32),
        ],
    )
    segsum = pl.kernel(
        _sc_segsum_body,
        mesh=mesh,
        out_type=[
            jax.ShapeDtypeStruct((NP, HALF), jnp.float32),
            jax.ShapeDtypeStruct((NP, HALF), jnp.float32),
        ],
        scratch_types=[
            pltpu.VMEM((CPW, CH), jnp.int32),
            pltpu.VMEM((CPW, CH), jnp.int32),
            pltpu.VMEM((CH, HALF), jnp.float32),
            pltpu.VMEM_SHARED((NP, HALF), jnp.float32),
            pltpu.SemaphoreType.DMA,
        ],
    )
    return degrees, segsum


def _norm_col(deg_ref):
    d = deg_ref[:, 0:1]
    return jnp.where(d > 0.0, 1.0 / jnp.sqrt(jnp.maximum(d, 1.0)), 0.0)


def _dot(a, b):
    return lax.dot_general(a, b, (((1,), (0,)), ((), ())),
                           precision=lax.Precision.HIGHEST,
                           preferred_element_type=jnp.float32)


def _mm1_body(x_ref, dego_ref, w_ref, out0_ref, out1_ref):
    ns = _norm_col(dego_ref)
    p = _dot(x_ref[...] * ns, w_ref[...])
    out0_ref[...] = p[:, :HALF]
    out1_ref[...] = p[:, HALF:]


def _mm2_body(a0_ref, a1_ref, degi_ref, dego_ref, b_ref, w_ref, out0_ref, out1_ref):
    nd = _norm_col(degi_ref)
    ns = _norm_col(dego_ref)
    h0 = jnp.tanh(a0_ref[...] * nd + b_ref[0:1, :]) * ns
    h1 = jnp.tanh(a1_ref[...] * nd + b_ref[1:2, :]) * ns
    w = w_ref[...]
    p = _dot(h0, w[:HALF, :]) + _dot(h1, w[HALF:, :])
    out0_ref[...] = p[:, :HALF]
    out1_ref[...] = p[:, HALF:]


def _final_body(a0_ref, a1_ref, degi_ref, b_ref, wc_ref, bc_ref, out_ref, acc_ref):
    i = pl.program_id(0)
    nd = _norm_col(degi_ref)
    h0 = jnp.tanh(a0_ref[...] * nd + b_ref[0:1, :])
    h1 = jnp.tanh(a1_ref[...] * nd + b_ref[1:2, :])
    row = lax.broadcasted_iota(jnp.int32, (BLK, 1), 0) + i * BLK
    m = (row < N).astype(jnp.float32)
    s0 = jnp.sum(h0 * m, axis=0, keepdims=True)
    s1 = jnp.sum(h1 * m, axis=0, keepdims=True)

    @pl.when(i == 0)
    def _():
        acc_ref[...] = jnp.zeros_like(acc_ref)

    acc_ref[0:1, :] += s0
    acc_ref[1:2, :] += s1

    @pl.when(i == NBLK - 1)
    def _():
        hg0 = jnp.tanh(acc_ref[0:1, :] * (1.0 / N))
        hg1 = jnp.tanh(acc_ref[1:2, :] * (1.0 / N))
        out_ref[...] = _dot(hg0, wc_ref[0, :, :]) + _dot(hg1, wc_ref[1, :, :]) + bc_ref[...]


def _mm1(x_pad, dego, W0):
    return pl.pallas_call(
        _mm1_body,
        grid=(NBLK,),
        in_specs=[
            pl.BlockSpec((BLK, D), lambda i: (i, 0)),
            pl.BlockSpec((BLK, HALF), lambda i: (i, 0)),
            pl.BlockSpec((D, D), lambda i: (0, 0)),
        ],
        out_specs=[pl.BlockSpec((BLK, HALF), lambda i: (i, 0))] * 2,
        out_shape=[jax.ShapeDtypeStruct((NP, HALF), jnp.float32)] * 2,
    )(x_pad, dego, W0)


def _mm2(a0, a1, degi, dego, br, W):
    return pl.pallas_call(
        _mm2_body,
        grid=(NBLK,),
        in_specs=[
            pl.BlockSpec((BLK, HALF), lambda i: (i, 0)),
            pl.BlockSpec((BLK, HALF), lambda i: (i, 0)),
            pl.BlockSpec((BLK, HALF), lambda i: (i, 0)),
            pl.BlockSpec((BLK, HALF), lambda i: (i, 0)),
            pl.BlockSpec((2, HALF), lambda i: (0, 0)),
            pl.BlockSpec((D, D), lambda i: (0, 0)),
        ],
        out_specs=[pl.BlockSpec((BLK, HALF), lambda i: (i, 0))] * 2,
        out_shape=[jax.ShapeDtypeStruct((NP, HALF), jnp.float32)] * 2,
    )(a0, a1, degi, dego, br, W)


def _final(a0, a1, degi, br, wc_r, bc_r):
    return pl.pallas_call(
        _final_body,
        grid=(NBLK,),
        in_specs=[
            pl.BlockSpec((BLK, HALF), lambda i: (i, 0)),
            pl.BlockSpec((BLK, HALF), lambda i: (i, 0)),
            pl.BlockSpec((BLK, HALF), lambda i: (i, 0)),
            pl.BlockSpec((2, HALF), lambda i: (0, 0)),
            pl.BlockSpec((2, HALF, HALF), lambda i: (0, 0, 0)),
            pl.BlockSpec((1, HALF), lambda i: (0, 0)),
        ],
        out_specs=pl.BlockSpec((1, HALF), lambda i: (0, 0)),
        out_shape=jax.ShapeDtypeStruct((1, HALF), jnp.float32),
        scratch_shapes=[pltpu.VMEM((2, HALF), jnp.float32)],
    )(a0, a1, degi, br, wc_r, bc_r)


def kernel(x, edge_index, W0, b0, W1, b1, W2, b2, Wc, bc):
    x_pad = jnp.pad(x, ((0, NP - N), (0, 0)))
    e32 = edge_index.reshape(2 * NSUB, CPW, CH)

    ones128 = jnp.ones((CH, HALF), jnp.float32)
    z128 = jnp.zeros((RPS, HALF), jnp.float32)

    sc_degrees, sc_segsum = _sc_kernels()
    deg2 = sc_degrees(e32, ones128, z128)
    dego, degi = deg2[0], deg2[1]

    b0r = b0.reshape(2, HALF)
    b1r = b1.reshape(2, HALF)
    b2r = b2.reshape(2, HALF)
    wc_r = jnp.pad(Wc, ((0, 0), (0, HALF - C_OUT))).reshape(2, HALF, HALF)
    bc_r = jnp.pad(bc, (0, HALF - C_OUT)).reshape(1, HALF)

    p0, p1 = _mm1(x_pad, dego, W0)
    a0, a1 = sc_segsum(e32, p0, p1, z128)
    p0, p1 = _mm2(a0, a1, degi, dego, b0r, W1)
    a0, a1 = sc_segsum(e32, p0, p1, z128)
    p0, p1 = _mm2(a0, a1, degi, dego, b1r, W2)
    a0, a1 = sc_segsum(e32, p0, p1, z128)

    out = _final(a0, a1, degi, b2r, wc_r, bc_r)
    return out[:, :C_OUT]


# speedup vs baseline: 6.0537x; 1.3381x over previous
import functools

import jax
import jax.numpy as jnp
from jax import lax
from jax.experimental import pallas as pl
from jax.experimental.pallas import tpu as pltpu
from jax.experimental.pallas import tpu_sc as plsc

N = 10000
NP = 10240
E = 160000
D = 256
HALF = 128
C_OUT = 8
BLK = 256
NBLK = NP // BLK
NSUB = 16
CH = 125
NPH = 2
CPW = E // (NSUB * CH)
RPS = NP // NSUB


def _sc_degrees_body(e_ref, ones_ref, z_ref, deg_ref, idxs_v, ones_v, deg_sh):
    c = lax.axis_index("c")
    s = lax.axis_index("s")
    pltpu.sync_copy(z_ref, deg_sh.at[pl.ds(s * RPS, RPS)])
    pltpu.sync_copy(ones_ref, ones_v)
    pltpu.sync_copy(e_ref.at[c * NSUB + s], idxs_v)
    plsc.subcore_barrier()

    def chunk(g, carry):
        pltpu.sync_copy(ones_v, deg_sh.at[idxs_v.at[g]], add=True)
        return carry

    lax.fori_loop(0, CPW, chunk, 0)
    plsc.subcore_barrier()
    sl = pl.ds(s * RPS, RPS)
    pltpu.sync_copy(deg_sh.at[sl], deg_ref.at[c].at[sl])


def _sc_segsum_body(e_ref, p0_ref, p1_ref, z_ref, out0_ref, out1_ref,
                    sidx_v, didx_v, rows0_v, rows1_v, agg_sh, sem0, sem1):
    c = lax.axis_index("c")
    s = lax.axis_index("s")
    pltpu.sync_copy(z_ref, agg_sh.at[pl.ds(s * RPS, RPS)])
    plsc.subcore_barrier()
    PH = CPW // NPH

    def run(p_ref):
        def gather(g, rows, sem):
            gw = jnp.where(g < PH, g, 0)
            pltpu.async_copy(p_ref.at[sidx_v.at[gw]], rows, sem)

        def gwait(g, rows, sem):
            gw = jnp.where(g < PH, g, 0)
            pltpu.make_async_copy(p_ref.at[sidx_v.at[gw]], rows, sem).wait()

        def scat(g, rows):
            pltpu.sync_copy(rows, agg_sh.at[didx_v.at[g]], add=True)

        def pair(k, carry):
            g = 2 * k
            gather(g + 1, rows1_v, sem1)
            gwait(g, rows0_v, sem0)
            scat(g, rows0_v)
            gather(g + 2, rows0_v, sem0)
            gwait(g + 1, rows1_v, sem1)
            scat(g + 1, rows1_v)
            return carry

        for ph in range(NPH):
            pltpu.sync_copy(e_ref.at[s, pl.ds(ph * PH, PH)], sidx_v)
            pltpu.sync_copy(e_ref.at[NSUB + s, pl.ds(ph * PH, PH)], didx_v)
            gather(0, rows0_v, sem0)
            lax.fori_loop(0, PH // 2, pair, 0)
            gwait(PH, rows0_v, sem0)

    pl.when(c == 0)(lambda: run(p0_ref))
    pl.when(c == 1)(lambda: run(p1_ref))
    plsc.subcore_barrier()
    sl = pl.ds(s * RPS, RPS)
    pl.when(c == 0)(lambda: pltpu.sync_copy(agg_sh.at[sl], out0_ref.at[sl]))
    pl.when(c == 1)(lambda: pltpu.sync_copy(agg_sh.at[sl], out1_ref.at[sl]))


@functools.cache
def _sc_kernels():
    mesh = plsc.VectorSubcoreMesh(core_axis_name="c", subcore_axis_name="s")
    degrees = pl.kernel(
        _sc_degrees_body,
        mesh=mesh,
        out_type=jax.ShapeDtypeStruct((2, NP, HALF), jnp.float32),
        scratch_types=[
            pltpu.VMEM((CPW, CH), jnp.int32),
            pltpu.VMEM((CH, HALF), jnp.float32),
            pltpu.VMEM_SHARED((NP, HALF), jnp.float32),
        ],
    )
    segsum = pl.kernel(
        _sc_segsum_body,
        mesh=mesh,
        out_type=[
            jax.ShapeDtypeStruct((NP, HALF), jnp.float32),
            jax.ShapeDtypeStruct((NP, HALF), jnp.float32),
        ],
        scratch_types=[
            pltpu.VMEM((CPW // NPH, CH), jnp.int32),
            pltpu.VMEM((CPW // NPH, CH), jnp.int32),
            pltpu.VMEM((CH, HALF), jnp.float32),
            pltpu.VMEM((CH, HALF), jnp.float32),
            pltpu.VMEM_SHARED((NP, HALF), jnp.float32),
            pltpu.SemaphoreType.DMA,
            pltpu.SemaphoreType.DMA,
        ],
    )
    return degrees, segsum


def _norm_col(deg_ref):
    d = deg_ref[:, 0:1]
    return jnp.where(d > 0.0, 1.0 / jnp.sqrt(jnp.maximum(d, 1.0)), 0.0)


def _dot(a, b):
    return lax.dot_general(a, b, (((1,), (0,)), ((), ())),
                           precision=lax.Precision.HIGHEST,
                           preferred_element_type=jnp.float32)


def _mm1_body(x_ref, dego_ref, w_ref, out0_ref, out1_ref):
    ns = _norm_col(dego_ref)
    p = _dot(x_ref[...] * ns, w_ref[...])
    out0_ref[...] = p[:, :HALF]
    out1_ref[...] = p[:, HALF:]


def _mm2_body(a0_ref, a1_ref, degi_ref, dego_ref, b_ref, w_ref, out0_ref, out1_ref):
    nd = _norm_col(degi_ref)
    ns = _norm_col(dego_ref)
    h0 = jnp.tanh(a0_ref[...] * nd + b_ref[0:1, :]) * ns
    h1 = jnp.tanh(a1_ref[...] * nd + b_ref[1:2, :]) * ns
    w = w_ref[...]
    p = _dot(h0, w[:HALF, :]) + _dot(h1, w[HALF:, :])
    out0_ref[...] = p[:, :HALF]
    out1_ref[...] = p[:, HALF:]


def _final_body(a0_ref, a1_ref, degi_ref, b_ref, wc_ref, bc_ref, out_ref, acc_ref):
    i = pl.program_id(0)
    nd = _norm_col(degi_ref)
    h0 = jnp.tanh(a0_ref[...] * nd + b_ref[0:1, :])
    h1 = jnp.tanh(a1_ref[...] * nd + b_ref[1:2, :])
    row = lax.broadcasted_iota(jnp.int32, (BLK, 1), 0) + i * BLK
    m = (row < N).astype(jnp.float32)
    s0 = jnp.sum(h0 * m, axis=0, keepdims=True)
    s1 = jnp.sum(h1 * m, axis=0, keepdims=True)

    @pl.when(i == 0)
    def _():
        acc_ref[...] = jnp.zeros_like(acc_ref)

    acc_ref[0:1, :] += s0
    acc_ref[1:2, :] += s1

    @pl.when(i == NBLK - 1)
    def _():
        hg0 = jnp.tanh(acc_ref[0:1, :] * (1.0 / N))
        hg1 = jnp.tanh(acc_ref[1:2, :] * (1.0 / N))
        out_ref[...] = _dot(hg0, wc_ref[0, :, :]) + _dot(hg1, wc_ref[1, :, :]) + bc_ref[...]


def _mm1(x_pad, dego, W0):
    return pl.pallas_call(
        _mm1_body,
        grid=(NBLK,),
        in_specs=[
            pl.BlockSpec((BLK, D), lambda i: (i, 0)),
            pl.BlockSpec((BLK, HALF), lambda i: (i, 0)),
            pl.BlockSpec((D, D), lambda i: (0, 0)),
        ],
        out_specs=[pl.BlockSpec((BLK, HALF), lambda i: (i, 0))] * 2,
        out_shape=[jax.ShapeDtypeStruct((NP, HALF), jnp.float32)] * 2,
    )(x_pad, dego, W0)


def _mm2(a0, a1, degi, dego, br, W):
    return pl.pallas_call(
        _mm2_body,
        grid=(NBLK,),
        in_specs=[
            pl.BlockSpec((BLK, HALF), lambda i: (i, 0)),
            pl.BlockSpec((BLK, HALF), lambda i: (i, 0)),
            pl.BlockSpec((BLK, HALF), lambda i: (i, 0)),
            pl.BlockSpec((BLK, HALF), lambda i: (i, 0)),
            pl.BlockSpec((2, HALF), lambda i: (0, 0)),
            pl.BlockSpec((D, D), lambda i: (0, 0)),
        ],
        out_specs=[pl.BlockSpec((BLK, HALF), lambda i: (i, 0))] * 2,
        out_shape=[jax.ShapeDtypeStruct((NP, HALF), jnp.float32)] * 2,
    )(a0, a1, degi, dego, br, W)


def _final(a0, a1, degi, br, wc_r, bc_r):
    return pl.pallas_call(
        _final_body,
        grid=(NBLK,),
        in_specs=[
            pl.BlockSpec((BLK, HALF), lambda i: (i, 0)),
            pl.BlockSpec((BLK, HALF), lambda i: (i, 0)),
            pl.BlockSpec((BLK, HALF), lambda i: (i, 0)),
            pl.BlockSpec((2, HALF), lambda i: (0, 0)),
            pl.BlockSpec((2, HALF, HALF), lambda i: (0, 0, 0)),
            pl.BlockSpec((1, HALF), lambda i: (0, 0)),
        ],
        out_specs=pl.BlockSpec((1, HALF), lambda i: (0, 0)),
        out_shape=jax.ShapeDtypeStruct((1, HALF), jnp.float32),
        scratch_shapes=[pltpu.VMEM((2, HALF), jnp.float32)],
    )(a0, a1, degi, br, wc_r, bc_r)


def kernel(x, edge_index, W0, b0, W1, b1, W2, b2, Wc, bc):
    x_pad = jnp.pad(x, ((0, NP - N), (0, 0)))
    e32 = edge_index.reshape(2 * NSUB, CPW, CH)

    ones128 = jnp.ones((CH, HALF), jnp.float32)
    z128 = jnp.zeros((RPS, HALF), jnp.float32)

    sc_degrees, sc_segsum = _sc_kernels()
    deg2 = sc_degrees(e32, ones128, z128)
    dego, degi = deg2[0], deg2[1]

    b0r = b0.reshape(2, HALF)
    b1r = b1.reshape(2, HALF)
    b2r = b2.reshape(2, HALF)
    wc_r = jnp.pad(Wc, ((0, 0), (0, HALF - C_OUT))).reshape(2, HALF, HALF)
    bc_r = jnp.pad(bc, (0, HALF - C_OUT)).reshape(1, HALF)

    p0, p1 = _mm1(x_pad, dego, W0)
    a0, a1 = sc_segsum(e32, p0, p1, z128)
    p0, p1 = _mm2(a0, a1, degi, dego, b0r, W1)
    a0, a1 = sc_segsum(e32, p0, p1, z128)
    p0, p1 = _mm2(a0, a1, degi, dego, b1r, W2)
    a0, a1 = sc_segsum(e32, p0, p1, z128)

    out = _final(a0, a1, degi, b2r, wc_r, bc_r)
    return out[:, :C_OUT]
